# baseline (speedup 1.0000x reference)
import numpy as np
import jax
import jax.numpy as jnp
from jax.experimental import pallas as pl

_BATCH = 512
_FEATURES = [64, 32, 16, 8]
_BRANCHES = [2, 8, 16]
_HC = 64
_T = 128
_G = _BATCH // _T

_INV = np.empty((256,), dtype=np.int32)
for _p in range(256):
    _c1 = _p // 128
    _cc1 = (_p // 16) % 8
    _cc2 = _p % 16
    _INV[_p] = _cc2 * 16 + _cc1 * 2 + _c1


def _dott(w, x):
    return jax.lax.dot_general(w, x, (((0,), (0,)), ((), ())),
                               preferred_element_type=jnp.float32)


def _row(v):
    return v.reshape(1, v.shape[0])


def _level_t(xT, C, f, fn, b, Wbr, bbr, ea, Wa1, ba1, Wa2, ba2, Was, bas,
             mpl):
    N = C * _T
    BN = b * N
    ones1 = jnp.ones((1, N), jnp.float32)
    onesB = jnp.ones((1, BN), jnp.float32)
    proj = _dott(jnp.concatenate([Wbr, _row(bbr)], axis=0),
                 jnp.concatenate([xT, ones1], axis=0))
    children = jnp.concatenate(
        [proj[cc * f:(cc + 1) * f, :] + xT for cc in range(b)], axis=1)
    src = jnp.concatenate([xT] * b, axis=1)
    eaRows = jnp.concatenate(
        [jnp.broadcast_to(ea[cc][:, None], (ea.shape[1], N))
         for cc in range(b)], axis=1)
    m_in = jnp.concatenate([src, children, eaRows, onesB], axis=0)
    h = jax.nn.relu(_dott(jnp.concatenate([Wa1, _row(ba1)], axis=0), m_in))
    bias2 = (bas + ba2).reshape(fn, 1)
    x = jax.nn.relu(_dott(Was, children) + _dott(Wa2, h) + bias2)
    for (Wm, bm, Wu, bu) in mpl:
        A = _dott(Wm[:fn], x)
        B = _dott(jnp.concatenate([Wm[fn:], _row(bm)], axis=0),
                  jnp.concatenate([x, onesB], axis=0))
        aggs = []
        for d in range(b):
            Bd = B[:, d * N:(d + 1) * N]
            a = None
            for s in range(b):
                if s == d:
                    continue
                term = jax.nn.relu(A[:, s * N:(s + 1) * N] + Bd)
                a = term if a is None else a + term
            aggs.append(a)
        agg = jnp.concatenate(aggs, axis=1)
        x = jax.nn.relu(_dott(Wu[:fn], x) + _dott(Wu[fn:], agg)
                        + bu.reshape(fn, 1))
    return x


def _body(*refs):
    rv_ref = refs[0]
    out_ref = refs[-1]
    pr = refs[1:-1]
    x = jnp.swapaxes(rv_ref[...], 0, 1)
    idx = 0
    C = 1
    for l in range(3):
        f, fn, b = _FEATURES[l], _FEATURES[l + 1], _BRANCHES[l]
        vals = [r[...] for r in pr[idx:idx + 17]]
        idx += 17
        (Wbr, bbr, ea, Wa1, ba1, Wa2, ba2, Was, bas) = vals[:9]
        mpl = [tuple(vals[9 + 4 * t:9 + 4 * (t + 1)]) for t in range(2)]
        x = _level_t(x, C, f, fn, b, Wbr, bbr, ea, Wa1, ba1, Wa2, ba2,
                     Was, bas, mpl)
        C *= b
    y = jnp.concatenate(
        [jnp.swapaxes(x[:, int(_INV[p]) * _T:(int(_INV[p]) + 1) * _T], 0, 1)
         for p in range(256)], axis=1)
    out_ref[...] = y


def _const_spec(shape):
    nd = len(shape)
    return pl.BlockSpec(shape, lambda i, _nd=nd: (0,) * _nd)


def kernel(random_vector, params):
    flat = []
    for l in range(3):
        p = params['lvl%d' % l]
        flat += [p['Wbr'], p['bbr'], p['ea'], p['Wa1'], p['ba1'],
                 p['Wa2'], p['ba2'], p['Was'], p['bas']]
        for t in range(2):
            m = p['mpl'][t]
            flat += [m['Wm'], m['bm'], m['Wu'], m['bu']]
    out = pl.pallas_call(
        _body,
        grid=(_G,),
        in_specs=[pl.BlockSpec((_T, _FEATURES[0]), lambda i: (i, 0))] +
                 [_const_spec(a.shape) for a in flat],
        out_specs=pl.BlockSpec((_T, 256 * _FEATURES[-1]),
                               lambda i: (i, 0)),
        out_shape=jax.ShapeDtypeStruct((_BATCH, 256 * _FEATURES[-1]),
                                       jnp.float32),
    )(random_vector, *flat)
    return out.reshape(_BATCH, 256, _FEATURES[-1])


# speedup vs baseline: 1.4850x; 1.4850x over previous
import numpy as np
import jax
import jax.numpy as jnp
from jax.experimental import pallas as pl

_BATCH = 512
_FEATURES = [64, 32, 16, 8]
_BRANCHES = [2, 8, 16]
_HC = 64
_T = 128
_G = _BATCH // _T

_INV = np.empty((256,), dtype=np.int32)
for _p in range(256):
    _c1 = _p // 128
    _cc1 = (_p // 16) % 8
    _cc2 = _p % 16
    _INV[_p] = _cc2 * 16 + _cc1 * 2 + _c1


def _dott(w, x):
    return jax.lax.dot_general(w, x, (((0,), (0,)), ((), ())),
                               preferred_element_type=jnp.float32)


def _row(v):
    return v.reshape(1, v.shape[0])


def _level_t(xT, C, f, fn, b, Wbr, bbr, ea, Wa1, ba1, Wa2, ba2, Was, bas,
             mpl):
    N = C * _T
    BN = b * N
    ones1 = jnp.ones((1, N), jnp.float32)
    onesB = jnp.ones((1, BN), jnp.float32)
    proj = _dott(jnp.concatenate([Wbr, _row(bbr)], axis=0),
                 jnp.concatenate([xT, ones1], axis=0))
    children = jnp.concatenate(
        [proj[cc * f:(cc + 1) * f, :] + xT for cc in range(b)], axis=1)
    src = jnp.concatenate([xT] * b, axis=1)
    eaRows = jnp.concatenate(
        [jnp.broadcast_to(ea[cc][:, None], (ea.shape[1], N))
         for cc in range(b)], axis=1)
    m_in = jnp.concatenate([src, children, eaRows, onesB], axis=0)
    h = jax.nn.relu(_dott(jnp.concatenate([Wa1, _row(ba1)], axis=0), m_in))
    bias2 = (bas + ba2).reshape(fn, 1)
    x = jax.nn.relu(_dott(Was, children) + _dott(Wa2, h) + bias2)
    for (Wm, bm, Wu, bu) in mpl:
        A = _dott(Wm[:fn], x)
        B = _dott(jnp.concatenate([Wm[fn:], _row(bm)], axis=0),
                  jnp.concatenate([x, onesB], axis=0))
        aggs = []
        for d in range(b):
            Bd = B[:, d * N:(d + 1) * N]
            a = None
            for s in range(b):
                if s == d:
                    continue
                term = jax.nn.relu(A[:, s * N:(s + 1) * N] + Bd)
                a = term if a is None else a + term
            aggs.append(a)
        agg = jnp.concatenate(aggs, axis=1)
        x = jax.nn.relu(_dott(Wu[:fn], x) + _dott(Wu[fn:], agg)
                        + bu.reshape(fn, 1))
    return x


def _body(*refs):
    rv_ref = refs[0]
    out_ref = refs[-1]
    pr = refs[1:-1]
    x = jnp.swapaxes(rv_ref[...], 0, 1)
    idx = 0
    C = 1
    for l in range(3):
        f, fn, b = _FEATURES[l], _FEATURES[l + 1], _BRANCHES[l]
        vals = [r[...] for r in pr[idx:idx + 17]]
        idx += 17
        (Wbr, bbr, ea, Wa1, ba1, Wa2, ba2, Was, bas) = vals[:9]
        mpl = [tuple(vals[9 + 4 * t:9 + 4 * (t + 1)]) for t in range(2)]
        x = _level_t(x, C, f, fn, b, Wbr, bbr, ea, Wa1, ba1, Wa2, ba2,
                     Was, bas, mpl)
        C *= b
    out_ref[...] = x


def _const_spec(shape):
    nd = len(shape)
    return pl.BlockSpec(shape, lambda i, _nd=nd: (0,) * _nd)


def kernel(random_vector, params):
    flat = []
    for l in range(3):
        p = params['lvl%d' % l]
        flat += [p['Wbr'], p['bbr'], p['ea'], p['Wa1'], p['ba1'],
                 p['Wa2'], p['ba2'], p['Was'], p['bas']]
        for t in range(2):
            m = p['mpl'][t]
            flat += [m['Wm'], m['bm'], m['Wu'], m['bu']]
    lanes_out = 256 * _T
    out = pl.pallas_call(
        _body,
        grid=(_G,),
        in_specs=[pl.BlockSpec((_T, _FEATURES[0]), lambda i: (i, 0))] +
                 [_const_spec(a.shape) for a in flat],
        out_specs=pl.BlockSpec((_FEATURES[-1], lanes_out),
                               lambda i: (0, i)),
        out_shape=jax.ShapeDtypeStruct((_FEATURES[-1], _G * lanes_out),
                                       jnp.float32),
    )(random_vector, *flat)
    o = out.reshape(_FEATURES[-1], _G, 16, 8, 2, _T)
    return o.transpose(1, 5, 4, 3, 2, 0).reshape(_BATCH, 256, _FEATURES[-1])


# speedup vs baseline: 1.4916x; 1.0044x over previous
import numpy as np
import jax
import jax.numpy as jnp
from jax.experimental import pallas as pl

_BATCH = 512
_FEATURES = [64, 32, 16, 8]
_BRANCHES = [2, 8, 16]
_HC = 64
_T = 128
_G = _BATCH // _T

_INV = np.empty((256,), dtype=np.int32)
for _p in range(256):
    _c1 = _p // 128
    _cc1 = (_p // 16) % 8
    _cc2 = _p % 16
    _INV[_p] = _cc2 * 16 + _cc1 * 2 + _c1


def _dott(w, x):
    return jax.lax.dot_general(w, x, (((0,), (0,)), ((), ())),
                               preferred_element_type=jnp.float32)


def _row(v):
    return v.reshape(1, v.shape[0])


def _level_t(xT, C, f, fn, b, Wbr, bbr, ea, Wa1, ba1, Wa2, ba2, Was, bas,
             mpl):
    N = C * _T
    BN = b * N
    ones1 = jnp.ones((1, N), jnp.float32)
    onesB = jnp.ones((1, BN), jnp.float32)
    proj = _dott(jnp.concatenate([Wbr, _row(bbr)], axis=0),
                 jnp.concatenate([xT, ones1], axis=0))
    children = jnp.concatenate(
        [proj[cc * f:(cc + 1) * f, :] + xT for cc in range(b)], axis=1)
    src = jnp.concatenate([xT] * b, axis=1)
    eaRows = jnp.concatenate(
        [jnp.broadcast_to(ea[cc][:, None], (ea.shape[1], N))
         for cc in range(b)], axis=1)
    m_in = jnp.concatenate([src, children, eaRows, onesB], axis=0)
    h = jax.nn.relu(_dott(jnp.concatenate([Wa1, _row(ba1)], axis=0), m_in))
    bias2 = (bas + ba2).reshape(fn, 1)
    x = jax.nn.relu(_dott(Was, children) + _dott(Wa2, h) + bias2)
    for (Wm, bm, Wu, bu) in mpl:
        A = _dott(Wm[:fn], x)
        B = _dott(jnp.concatenate([Wm[fn:], _row(bm)], axis=0),
                  jnp.concatenate([x, onesB], axis=0))
        buC = bu.reshape(fn, 1)
        x_parts = []
        for d in range(b):
            Bd = B[:, d * N:(d + 1) * N]
            a = None
            for s in range(b):
                if s == d:
                    continue
                term = jax.nn.relu(A[:, s * N:(s + 1) * N] + Bd)
                a = term if a is None else a + term
            xd = x[:, d * N:(d + 1) * N]
            x_parts.append(jax.nn.relu(
                _dott(Wu[:fn], xd) + _dott(Wu[fn:], a) + buC))
        x = jnp.concatenate(x_parts, axis=1)
    return x


def _body(*refs):
    rv_ref = refs[0]
    out_ref = refs[-1]
    pr = refs[1:-1]
    x = jnp.swapaxes(rv_ref[...], 0, 1)
    idx = 0
    C = 1
    for l in range(3):
        f, fn, b = _FEATURES[l], _FEATURES[l + 1], _BRANCHES[l]
        vals = [r[...] for r in pr[idx:idx + 17]]
        idx += 17
        (Wbr, bbr, ea, Wa1, ba1, Wa2, ba2, Was, bas) = vals[:9]
        mpl = [tuple(vals[9 + 4 * t:9 + 4 * (t + 1)]) for t in range(2)]
        x = _level_t(x, C, f, fn, b, Wbr, bbr, ea, Wa1, ba1, Wa2, ba2,
                     Was, bas, mpl)
        C *= b
    out_ref[...] = x


def _const_spec(shape):
    nd = len(shape)
    return pl.BlockSpec(shape, lambda i, _nd=nd: (0,) * _nd)


def kernel(random_vector, params):
    flat = []
    for l in range(3):
        p = params['lvl%d' % l]
        flat += [p['Wbr'], p['bbr'], p['ea'], p['Wa1'], p['ba1'],
                 p['Wa2'], p['ba2'], p['Was'], p['bas']]
        for t in range(2):
            m = p['mpl'][t]
            flat += [m['Wm'], m['bm'], m['Wu'], m['bu']]
    lanes_out = 256 * _T
    out = pl.pallas_call(
        _body,
        grid=(_G,),
        in_specs=[pl.BlockSpec((_T, _FEATURES[0]), lambda i: (i, 0))] +
                 [_const_spec(a.shape) for a in flat],
        out_specs=pl.BlockSpec((_FEATURES[-1], lanes_out),
                               lambda i: (0, i)),
        out_shape=jax.ShapeDtypeStruct((_FEATURES[-1], _G * lanes_out),
                                       jnp.float32),
    )(random_vector, *flat)
    o = out.reshape(_FEATURES[-1], _G, 16, 8, 2, _T)
    return o.transpose(1, 5, 4, 3, 2, 0).reshape(_BATCH, 256, _FEATURES[-1])
